# baseline (speedup 1.0000x reference)
import functools

import jax
import jax.numpy as jnp
from jax import lax
from jax.experimental import pallas as pl
from jax.experimental.pallas import tpu as pltpu
from jax.experimental.pallas import tpu_sc as plsc

_NUM_SAMPLES = 100000
_NUM_SGS = 24
_BATCH = 16384
_K = 4
_D = 2


def _make_gather():
    info = plsc.get_sparse_core_info()
    nc, ns = info.num_cores, info.num_subcores
    nw = nc * ns
    b_per_w = _BATCH // nw
    n_outer = b_per_w // _K
    mesh = plsc.VectorSubcoreMesh(core_axis_name="c", subcore_axis_name="s")

    @functools.partial(
        pl.kernel,
        mesh=mesh,
        out_type=jax.ShapeDtypeStruct((_BATCH, _NUM_SGS, 7), jnp.float32),
        scratch_types=(
            [pltpu.VMEM((b_per_w + _K,), jnp.int32)]
            + [pltpu.VMEM((1, _NUM_SGS, 7), jnp.float32) for _ in range(_K)]
            + [pltpu.SemaphoreType.DMA for _ in range(_K)]
        ),
    )
    def gather_kernel(idx_hbm, table_hbm, out_hbm, idx_v, *rest):
        bufs = rest[:_K]
        gsem = rest[_K:]
        wid = lax.axis_index("s") * nc + lax.axis_index("c")
        base = wid * b_per_w
        pltpu.sync_copy(
            idx_hbm.at[pl.ds(base, b_per_w)], idx_v.at[pl.ds(0, b_per_w)]
        )

        def issue_gather(r, slot):
            pltpu.async_copy(
                table_hbm.at[pl.ds(r, 1)], bufs[slot], gsem[slot]
            )

        def wait_gather(slot):
            pltpu.make_async_copy(
                table_hbm.at[pl.ds(0, 1)], bufs[slot], gsem[slot]
            ).wait()

        v0 = idx_v[pl.ds(0, _K)]
        for i in range(_D):
            issue_gather(v0[i], i)

        def outer(o, carry):
            i0 = o * _K
            v = idx_v[pl.ds(i0 + _D, _K)]
            for b in range(_K):
                i = i0 + b

                @pl.when(i + _D < b_per_w)
                def _():
                    issue_gather(v[b], (b + _D) % _K)

                wait_gather(b)
                pltpu.sync_copy(bufs[b], out_hbm.at[pl.ds(base + i, 1)])
            return carry

        lax.fori_loop(0, n_outer, outer, 0, unroll=False)

    return gather_kernel


_GATHER = _make_gather()


def kernel(idxs, sgs):
    return _GATHER(idxs.astype(jnp.int32), sgs)


# speedup vs baseline: 1.0754x; 1.0754x over previous
import functools

import jax
import jax.numpy as jnp
from jax import lax
from jax.experimental import pallas as pl
from jax.experimental.pallas import tpu as pltpu
from jax.experimental.pallas import tpu_sc as plsc

_NUM_SAMPLES = 100000
_NUM_SGS = 24
_BATCH = 16384
_G = 16


def _make_gather():
    info = plsc.get_sparse_core_info()
    nc, ns = info.num_cores, info.num_subcores
    nw = nc * ns
    b_per_w = _BATCH // nw
    n_chunks = b_per_w // _G
    n_outer = n_chunks // 2
    mesh = plsc.VectorSubcoreMesh(core_axis_name="c", subcore_axis_name="s")

    @functools.partial(
        pl.kernel,
        mesh=mesh,
        out_type=jax.ShapeDtypeStruct((_BATCH, _NUM_SGS, 7), jnp.float32),
        scratch_types=[
            pltpu.VMEM((b_per_w,), jnp.int32),
            pltpu.VMEM((_G, _NUM_SGS, 7), jnp.float32),
            pltpu.VMEM((_G, _NUM_SGS, 7), jnp.float32),
            pltpu.SemaphoreType.DMA,
            pltpu.SemaphoreType.DMA,
        ],
    )
    def gather_kernel(idx_hbm, table_hbm, out_hbm, idx_v, buf_a, buf_b, sem_a, sem_b):
        bufs = (buf_a, buf_b)
        sems = (sem_a, sem_b)
        wid = lax.axis_index("s") * nc + lax.axis_index("c")
        base = wid * b_per_w
        pltpu.sync_copy(idx_hbm.at[pl.ds(base, b_per_w)], idx_v)

        def fire(c, p):
            v = idx_v[pl.ds(c * _G, _G)]
            for j in range(_G):
                pltpu.async_copy(
                    table_hbm.at[pl.ds(v[j], 1)],
                    bufs[p].at[pl.ds(j, 1)],
                    sems[p],
                )

        def drain_write(c, p):
            pltpu.make_async_copy(
                table_hbm.at[pl.ds(0, _G)], bufs[p], sems[p]
            ).wait()
            pltpu.sync_copy(bufs[p], out_hbm.at[pl.ds(base + c * _G, _G)])

        fire(0, 0)

        def outer(o, carry):
            c0 = o * 2
            fire(c0 + 1, 1)
            drain_write(c0, 0)

            @pl.when(c0 + 2 < n_chunks)
            def _():
                fire(c0 + 2, 0)

            drain_write(c0 + 1, 1)
            return carry

        lax.fori_loop(0, n_outer, outer, 0, unroll=False)

    return gather_kernel


_GATHER = _make_gather()


def kernel(idxs, sgs):
    return _GATHER(idxs.astype(jnp.int32), sgs)
